# Initial kernel scaffold; baseline (speedup 1.0000x reference)
import jax
import jax.numpy as jnp
from jax.experimental import pallas as pl


def kernel(inputs, W):
    raise NotImplementedError("write your pallas kernel here")


# speedup vs baseline: 4.1126x; 4.1126x over previous
import functools

import jax
import jax.numpy as jnp
from jax import lax
from jax.experimental import pallas as pl
from jax.experimental.pallas import tpu as pltpu
from jax.experimental.pallas import tpu_sc as plsc

B, L, UNITS = 4096, 200, 64
N = B * L
NC, NS = 2, 16
NW = NC * NS
PER_W = N // NW
C = 512
N_CHUNKS = PER_W // C


def _body(x_hbm, w_hbm, out_hbm, wt, x_v, idx_v, al_v, out_v):
    wid = lax.axis_index("s") * NC + lax.axis_index("c")
    pltpu.sync_copy(w_hbm, wt)

    def chunk_body(ci, _):
        base = wid * PER_W + ci * C
        pltpu.sync_copy(x_hbm.at[pl.ds(base, C)], x_v)

        def vec_body(g, _):
            v = x_v[pl.ds(g * 16, 16)]
            iv = v.astype(jnp.int32)
            idx_v[pl.ds(g * 16, 16)] = iv * UNITS
            al_v[pl.ds(g * 16, 16)] = v - iv.astype(jnp.float32)
            return 0

        lax.fori_loop(0, C // 16, vec_body, 0, unroll=2)

        def lerp_body(g, _):
            offv = idx_v[pl.ds(g * 16, 16)]
            alv = al_v[pl.ds(g * 16, 16)]
            for lane in range(16):
                off = offv[lane]
                av = jnp.full((16,), alv[lane], jnp.float32)
                out_base = (g * 16 + lane) * UNITS
                for j in range(4):
                    lo = wt[pl.ds(off + 16 * j, 16)]
                    hi = wt[pl.ds(off + UNITS + 16 * j, 16)]
                    out_v[pl.ds(out_base + 16 * j, 16)] = lo + av * (hi - lo)
            return 0

        lax.fori_loop(0, C // 16, lerp_body, 0)
        pltpu.sync_copy(out_v, out_hbm.at[pl.ds(base * UNITS, C * UNITS)])
        return 0

    lax.fori_loop(0, N_CHUNKS, chunk_body, 0)


@jax.jit
def _run(x_flat, w_flat):
    mesh = plsc.VectorSubcoreMesh(core_axis_name="c", subcore_axis_name="s")
    return pl.kernel(
        _body,
        out_type=jax.ShapeDtypeStruct((N * UNITS,), jnp.float32),
        mesh=mesh,
        scratch_types=[
            pltpu.VMEM((UNITS * 256,), jnp.float32),
            pltpu.VMEM((C,), jnp.float32),
            pltpu.VMEM((C,), jnp.int32),
            pltpu.VMEM((C,), jnp.float32),
            pltpu.VMEM((C * UNITS,), jnp.float32),
        ],
    )(x_flat, w_flat)


def kernel(inputs, W):
    x_flat = inputs.reshape(N)
    w_flat = W.reshape(256 * UNITS)
    out = _run(x_flat, w_flat)
    return out.reshape(B, L, 1, UNITS)


# speedup vs baseline: 4.1421x; 1.0072x over previous
import functools

import jax
import jax.numpy as jnp
from jax import lax
from jax.experimental import pallas as pl
from jax.experimental.pallas import tpu as pltpu
from jax.experimental.pallas import tpu_sc as plsc

B, L, UNITS = 4096, 200, 64
N = B * L
NC, NS = 2, 16
NW = NC * NS
PER_W = N // NW
C = 512
N_CHUNKS = PER_W // C


def _body(x_hbm, w_hbm, out_hbm, wt, x_v, out_v):
    wid = lax.axis_index("s") * NC + lax.axis_index("c")
    pltpu.sync_copy(w_hbm, wt)

    iota = lax.iota(jnp.int32, 16)

    def chunk_body(ci, _):
        base = wid * PER_W + ci * C
        pltpu.sync_copy(x_hbm.at[pl.ds(base, C)], x_v)

        def lerp_body(g, _):
            v = x_v[pl.ds(g * 16, 16)]
            iv = v.astype(jnp.int32)
            alv = v - iv.astype(jnp.float32)
            for lane in range(16):
                row = jnp.full((16,), iv[lane], jnp.int32)
                av = jnp.full((16,), alv[lane], jnp.float32)
                out_base = (g * 16 + lane) * UNITS
                for j in range(4):
                    col = iota + 16 * j
                    lo = plsc.load_gather(wt, [row, col])
                    hi = plsc.load_gather(wt, [row + 1, col])
                    out_v[pl.ds(out_base + 16 * j, 16)] = lo + av * (hi - lo)
            return 0

        lax.fori_loop(0, C // 16, lerp_body, 0)
        pltpu.sync_copy(out_v, out_hbm.at[pl.ds(base * UNITS, C * UNITS)])
        return 0

    lax.fori_loop(0, N_CHUNKS, chunk_body, 0)


@jax.jit
def _run(x_flat, w_flat):
    mesh = plsc.VectorSubcoreMesh(core_axis_name="c", subcore_axis_name="s")
    return pl.kernel(
        _body,
        out_type=jax.ShapeDtypeStruct((N * UNITS,), jnp.float32),
        mesh=mesh,
        compiler_params=pltpu.CompilerParams(needs_layout_passes=False),
        scratch_types=[
            pltpu.VMEM((256, UNITS), jnp.float32),
            pltpu.VMEM((C,), jnp.float32),
            pltpu.VMEM((C * UNITS,), jnp.float32),
        ],
    )(x_flat, w_flat)


def kernel(inputs, W):
    x_flat = inputs.reshape(N)
    out = _run(x_flat, W)
    return out.reshape(B, L, 1, UNITS)


# speedup vs baseline: 5.7406x; 1.3859x over previous
import functools

import jax
import jax.numpy as jnp
from jax import lax
from jax.experimental import pallas as pl
from jax.experimental.pallas import tpu as pltpu
from jax.experimental.pallas import tpu_sc as plsc

B, L, UNITS = 4096, 200, 64
N = B * L
NC, NS = 2, 16
NW = NC * NS
PER_W = N // NW
C = 512
N_CHUNKS = PER_W // C


def _body(x_hbm, w_hbm, out_hbm, wt, x_v, out_v):
    wid = lax.axis_index("s") * NC + lax.axis_index("c")
    pltpu.sync_copy(w_hbm, wt)

    iota = lax.iota(jnp.int32, 16)

    def chunk_body(ci, _):
        base = wid * PER_W + ci * C
        pltpu.sync_copy(x_hbm.at[pl.ds(base, C)], x_v)

        @plsc.parallel_loop(0, C // 16, unroll=2)
        def lerp_body(g):
            v = x_v[pl.ds(g * 16, 16)]
            iv = v.astype(jnp.int32)
            alv = v - iv.astype(jnp.float32)
            for lane in range(16):
                row = jnp.full((16,), iv[lane], jnp.int32)
                av = jnp.full((16,), alv[lane], jnp.float32)
                out_base = (g * 16 + lane) * UNITS
                for j in range(4):
                    col = iota + 16 * j
                    lo = plsc.load_gather(wt, [row, col])
                    hi = plsc.load_gather(wt, [row + 1, col])
                    out_v[pl.ds(out_base + 16 * j, 16)] = lo + av * (hi - lo)
        pltpu.sync_copy(out_v, out_hbm.at[pl.ds(base * UNITS, C * UNITS)])
        return 0

    lax.fori_loop(0, N_CHUNKS, chunk_body, 0)


@jax.jit
def _run(x_flat, w_flat):
    mesh = plsc.VectorSubcoreMesh(core_axis_name="c", subcore_axis_name="s")
    return pl.kernel(
        _body,
        out_type=jax.ShapeDtypeStruct((N * UNITS,), jnp.float32),
        mesh=mesh,
        compiler_params=pltpu.CompilerParams(needs_layout_passes=False),
        scratch_types=[
            pltpu.VMEM((256, UNITS), jnp.float32),
            pltpu.VMEM((C,), jnp.float32),
            pltpu.VMEM((C * UNITS,), jnp.float32),
        ],
    )(x_flat, w_flat)


def kernel(inputs, W):
    x_flat = inputs.reshape(N)
    out = _run(x_flat, W)
    return out.reshape(B, L, 1, UNITS)
